# baseline (speedup 1.0000x reference)
import functools

import jax
import jax.numpy as jnp
from jax import lax
from jax.experimental import pallas as pl
from jax.experimental.pallas import tpu as pltpu
from jax.experimental.pallas import tpu_sc as plsc

N = 8388608
TW = 2 ** 19
NC, NS, L = 2, 16, 16
NW = NC * NS
KPT = N // NW
CHUNK = 16384
NCHUNK = KPT // CHUNK
GROUPS = CHUNK // 64
TAB_SLICE = TW // NS

_mesh = plsc.VectorSubcoreMesh(
    core_axis_name="c", subcore_axis_name="s", num_cores=NC, num_subcores=NS)


@functools.partial(
    pl.kernel,
    out_type=jax.ShapeDtypeStruct((N // 4 // 128, 128), jnp.int32),
    mesh=_mesh,
    scratch_types=[
        pltpu.VMEM_SHARED((TW,), jnp.int32),
        pltpu.VMEM((CHUNK,), jnp.int32),
        pltpu.VMEM((CHUNK,), jnp.int32),
        pltpu.VMEM((CHUNK,), jnp.int32),
        pltpu.VMEM((CHUNK,), jnp.int32),
        pltpu.VMEM((CHUNK // 4 // 128, 128), jnp.int32),
    ],
    compiler_params=pltpu.CompilerParams(
        needs_layout_passes=False, use_tc_tiling_on_sc=True),
)
def _lcg_sc(x_hbm, tab_hbm, out8_hbm, tab_sh, xbuf, wqbuf, bpbuf, twbuf, obuf):
    cid = lax.axis_index("c")
    sid = lax.axis_index("s")
    wid = sid * NC + cid


    for st in range(TAB_SLICE // CHUNK):
        woff = pl.multiple_of(sid * jnp.int32(TAB_SLICE) + jnp.int32(st * CHUNK), 8)
        pltpu.sync_copy(tab_hbm.at[pl.ds(woff, CHUNK)], xbuf)
        pltpu.sync_copy(xbuf, tab_sh.at[pl.ds(woff, CHUNK)])
    plsc.subcore_barrier()

    lanes4 = lax.iota(jnp.int32, L) * 4
    c7 = jnp.full((L,), 7, jnp.int32)
    c5 = jnp.full((L,), 5, jnp.int32)
    c31 = jnp.full((L,), 31, jnp.int32)
    c1 = jnp.full((L,), 1, jnp.int32)

    def chunk_body(g, carry):
        kbase = pl.multiple_of(wid * jnp.int32(KPT) + g * jnp.int32(CHUNK), 8)
        xrow = kbase // jnp.int32(CHUNK)
        pltpu.sync_copy(x_hbm.at[xrow, :], xbuf)

        def idx_body(t, c2):
            o = t * jnp.int32(64)
            for j in range(4):
                hi = plsc.load_gather(xbuf, [lanes4 + (o + jnp.int32(j))])
                i24 = lax.shift_right_logical(hi, c7)
                wqbuf[pl.ds(o + jnp.int32(j * 16), L)] = lax.shift_right_logical(i24, c5)
                bpbuf[pl.ds(o + jnp.int32(j * 16), L)] = lax.bitwise_and(i24, c31)
            return c2
        lax.fori_loop(jnp.int32(0), jnp.int32(GROUPS), idx_body, jnp.int32(0), unroll=False)

        pltpu.sync_copy(tab_sh.at[wqbuf], twbuf)

        def pack_body(t, c2):
            o = t * jnp.int32(64)
            acc = jnp.zeros((L,), jnp.int32)
            for j in range(4):
                tw = twbuf[pl.ds(o + jnp.int32(j * 16), L)]
                bp = bpbuf[pl.ds(o + jnp.int32(j * 16), L)]
                bit = lax.bitwise_and(lax.shift_right_logical(tw, bp), c1)
                if j:
                    bit = lax.shift_left(bit, jnp.full((L,), 8 * j, jnp.int32))
                acc = lax.bitwise_or(acc, bit)
            obuf[t >> 3, pl.ds((t & jnp.int32(7)) * jnp.int32(16), L)] = acc
            return c2
        lax.fori_loop(jnp.int32(0), jnp.int32(GROUPS), pack_body, jnp.int32(0), unroll=False)

        orow = pl.multiple_of(kbase // jnp.int32(4 * 128), 8)
        pltpu.sync_copy(obuf, out8_hbm.at[pl.ds(orow, CHUNK // 4 // 128), :])
        return carry

    lax.fori_loop(jnp.int32(0), jnp.int32(NCHUNK), chunk_body, jnp.int32(0), unroll=False)


def kernel(x, is_training, test_local_stats, binary_set):
    xhi = lax.shift_right_logical(x, 32).astype(jnp.int32)
    x32 = xhi.reshape(N // CHUNK, CHUNK)
    tab = lax.bitcast_convert_type(
        binary_set.reshape(TW // 128, 128, 4), jnp.int32).reshape(TW)
    outw = _lcg_sc(x32, tab)
    outb = lax.bitcast_convert_type(outw, jnp.uint8)
    return outb.reshape(N).astype(jnp.bool_)
